# baseline (speedup 1.0000x reference)
import functools

import jax
import jax.numpy as jnp
from jax import lax
from jax.experimental import pallas as pl
from jax.experimental.pallas import tpu as pltpu
from jax.experimental.pallas import tpu_sc as plsc

N = 10000
E = 320000
DF = 128
GAMMA = 1.0
STEPS = 14

NCORES = 2
NTILES = 16
CHUNK = 128
SEG = 16
SEGE = SEG * CHUNK
E_PAD = ((E + CHUNK - 1) // CHUNK) * CHUNK
NWIN = E_PAD // CHUNK
E_STORE = E_PAD + SEGE
ROWS_PT = 640
LAST_ROWS = N - 15 * ROWS_PT

_mesh = plsc.VectorSubcoreMesh(core_axis_name="c", subcore_axis_name="s")


@functools.partial(
    pl.kernel,
    out_type=(
        jax.ShapeDtypeStruct((N, DF), jnp.float32),
        jax.ShapeDtypeStruct((N, DF), jnp.float32),
    ),
    mesh=_mesh,
    scratch_types=[
        pltpu.VMEM((ROWS_PT, DF), jnp.float32),
        pltpu.VMEM((CHUNK, DF), jnp.float32),
        pltpu.VMEM((CHUNK, DF), jnp.float32),
        pltpu.VMEM((SEGE,), jnp.int32),
        pltpu.VMEM((SEGE,), jnp.int32),
        pltpu.VMEM((SEGE,), jnp.float32),
        pltpu.VMEM((16,), jnp.int32),
        pltpu.SemaphoreType.DMA,
        pltpu.SemaphoreType.DMA,
    ],
)
def _spmm_pair(x0, x1, src_h, dst_h, w0_h, w1_h, bounds_h, o0, o1,
               acc, rows0, rows1, srcb, dstb, wb, bv, gsem0, gsem1):
    c = lax.axis_index("c")
    s = lax.axis_index("s")
    lo_row = s * ROWS_PT

    pltpu.sync_copy(bounds_h.at[pl.ds(s * 16, 16)], bv)
    bvec = bv[pl.ds(0, 16)]
    clo = bvec[0]
    cnt = bvec[8]

    zero16 = jnp.zeros((16,), jnp.float32)

    def zrow(r, carry):
        for k8 in range(8):
            acc[r, pl.ds(k8 * 16, 16)] = zero16
        return carry

    lax.fori_loop(0, ROWS_PT, zrow, 0)

    def gather(j, buf, sem):
        idx = srcb.at[pl.ds(j * CHUNK, CHUNK)]

        @pl.when(c == 0)
        def _():
            pltpu.async_copy(x0.at[idx], buf, sem)

        @pl.when(c == 1)
        def _():
            pltpu.async_copy(x1.at[idx], buf, sem)

    def gather_wait(buf, sem):
        pltpu.make_async_copy(x0.at[srcb.at[pl.ds(0, CHUNK)]], buf, sem).wait()

    def scale_acc(buf, wloc):
        def group(g, gcarry):
            off = wloc * CHUNK + g * 16
            wvec = wb[pl.ds(off, 16)]
            dvec = dstb[pl.ds(off, 16)]
            dloc = dvec - lo_row
            valid = jnp.logical_and(dloc >= 0, dloc < ROWS_PT)
            wm = jnp.where(valid, wvec, 0.0)
            dcl = jnp.clip(dloc, 0, ROWS_PT - 1)
            for jj in range(16):
                e = g * 16 + jj
                w = wm[jj]
                dl = dcl[jj]
                for k8 in range(8):
                    sl = pl.ds(k8 * 16, 16)
                    plsc.addupdate(acc.at[dl, sl], buf[e, sl] * w)
            return gcarry

        lax.fori_loop(0, CHUNK // 16, group, 0)

    nblk = (cnt + SEG - 1) // SEG

    def block_body(b, carry):
        bstart = clo + b * SEG
        bcnt = jnp.minimum(cnt - b * SEG, SEG)
        e0 = bstart * CHUNK
        pltpu.sync_copy(src_h.at[pl.ds(e0, SEGE)], srcb)
        pltpu.sync_copy(dst_h.at[pl.ds(e0, SEGE)], dstb)

        @pl.when(c == 0)
        def _():
            pltpu.sync_copy(w0_h.at[pl.ds(e0, SEGE)], wb)

        @pl.when(c == 1)
        def _():
            pltpu.sync_copy(w1_h.at[pl.ds(e0, SEGE)], wb)

        gather(0, rows0, gsem0)

        def pair_body(g, pcarry):
            w0l = 2 * g

            @pl.when(w0l + 1 < bcnt)
            def _():
                gather(w0l + 1, rows1, gsem1)

            gather_wait(rows0, gsem0)
            scale_acc(rows0, w0l)

            @pl.when(w0l + 2 < bcnt)
            def _():
                gather(w0l + 2, rows0, gsem0)

            @pl.when(w0l + 1 < bcnt)
            def _():
                gather_wait(rows1, gsem1)
                scale_acc(rows1, w0l + 1)

            return pcarry

        lax.fori_loop(0, (bcnt + 1) // 2, pair_body, 0)
        return carry

    lax.fori_loop(0, nblk, block_body, 0)

    @pl.when(jnp.logical_and(c == 0, s < NTILES - 1))
    def _():
        pltpu.sync_copy(acc, o0.at[pl.ds(lo_row, ROWS_PT)])

    @pl.when(jnp.logical_and(c == 1, s < NTILES - 1))
    def _():
        pltpu.sync_copy(acc, o1.at[pl.ds(lo_row, ROWS_PT)])

    @pl.when(jnp.logical_and(c == 0, s == NTILES - 1))
    def _():
        pltpu.sync_copy(acc.at[pl.ds(0, LAST_ROWS)], o0.at[pl.ds(lo_row, LAST_ROWS)])

    @pl.when(jnp.logical_and(c == 1, s == NTILES - 1))
    def _():
        pltpu.sync_copy(acc.at[pl.ds(0, LAST_ROWS)], o1.at[pl.ds(lo_row, LAST_ROWS)])


def _soft(x, eta):
    return jax.nn.relu(x - eta) - jax.nn.relu(-x - eta)


def kernel(F, edge_index, w0_values, w1_values, d, mask):
    dst = edge_index[0]
    src = edge_index[1]
    npad = E_PAD - E
    padidx = (jnp.arange(npad, dtype=jnp.int32) * 97) % N
    src_p = jnp.concatenate([src, padidx])
    dst_p = jnp.concatenate([dst, padidx])
    zpad = jnp.zeros((npad,), jnp.float32)
    w0_p = jnp.concatenate([w0_values, zpad])
    w1_p = jnp.concatenate([w1_values, zpad])

    bucket = dst_p // ROWS_PT
    order = jnp.argsort(bucket)
    src_s = src_p[order]
    dst_s = dst_p[order]
    w0_s = w0_p[order]
    w1_s = w1_p[order]
    counts = jnp.bincount(bucket, length=NTILES)
    ends = jnp.cumsum(counts)
    starts = ends - counts
    clo = (starts // CHUNK).astype(jnp.int32)
    chi = ((ends + CHUNK - 1) // CHUNK).astype(jnp.int32)
    cnt = chi - clo
    bounds = jnp.repeat(jnp.stack([clo, cnt], axis=1), 8, axis=1).reshape(-1)

    tail = E_STORE - E_PAD
    src_s = jnp.concatenate([src_s, jnp.zeros((tail,), jnp.int32)])
    dst_s = jnp.concatenate([dst_s, jnp.zeros((tail,), jnp.int32)])
    w0_s = jnp.concatenate([w0_s, jnp.zeros((tail,), jnp.float32)])
    w1_s = jnp.concatenate([w1_s, jnp.zeros((tail,), jnp.float32)])

    def spmm_pair(X0, X1):
        return _spmm_pair(X0, X1, src_s, dst_s, w0_s, w1_s, bounds)

    d1 = d[:, None]
    m2 = mask * mask
    c2 = 1.0 / (d1 * m2 + GAMMA)
    c1F = d1 * m2 * F

    A0, A1 = spmm_pair(F, F)
    Y1 = jnp.zeros((N, DF), jnp.float32)
    U = F
    for k in range(1, STEPS + 1):
        v1 = Y1 - _soft(A1 + Y1, d1)
        P0, P1 = spmm_pair(A0, v1)
        U = (c1F - P1 + P0) * c2
        if k < STEPS:
            B0, B1 = spmm_pair(U, U)
            Y1 = B1 + v1
            A0, A1 = B0, B1
    return U


# speedup vs baseline: 1.4974x; 1.4974x over previous
import functools

import jax
import jax.numpy as jnp
from jax import lax
from jax.experimental import pallas as pl
from jax.experimental.pallas import tpu as pltpu
from jax.experimental.pallas import tpu_sc as plsc

N = 10000
E = 320000
DF = 128
GAMMA = 1.0
STEPS = 14

NCORES = 2
NTILES = 16
CHUNK = 128
NCHUNKS = 160
SEG = 16
NBLK = NCHUNKS // SEG
EPT = NCHUNKS * CHUNK
E_PAD = EPT * NTILES
ROWS_PT = 640
ACC_N = NTILES * ROWS_PT
LAST_ROWS = N - 15 * ROWS_PT

_mesh = plsc.VectorSubcoreMesh(core_axis_name="c", subcore_axis_name="s")


@functools.partial(
    pl.kernel,
    out_type=(
        jax.ShapeDtypeStruct((N, DF), jnp.float32),
        jax.ShapeDtypeStruct((N, DF), jnp.float32),
    ),
    mesh=_mesh,
    scratch_types=[
        pltpu.VMEM_SHARED((ACC_N, DF), jnp.float32),
        pltpu.VMEM((CHUNK, DF), jnp.float32),
        pltpu.VMEM((CHUNK, DF), jnp.float32),
        pltpu.VMEM((SEG, CHUNK), jnp.int32),
        pltpu.VMEM((SEG, CHUNK), jnp.int32),
        pltpu.VMEM((SEG, CHUNK), jnp.float32),
        pltpu.SemaphoreType.DMA,
        pltpu.SemaphoreType.DMA,
        pltpu.SemaphoreType.DMA,
        pltpu.SemaphoreType.DMA,
    ],
)
def _spmm_pair(x0, x1, src_h, dst_h, w0_h, w1_h, o0, o1,
               acc, rows0, rows1, srcv, dstv, wv, gsem0, gsem1, ssem0, ssem1):
    c = lax.axis_index("c")
    s = lax.axis_index("s")
    t0 = s * NCHUNKS

    zero16 = jnp.zeros((16,), jnp.float32)

    def zrow(r, carry):
        for k8 in range(8):
            rows0[r, pl.ds(k8 * 16, 16)] = zero16
        return carry

    lax.fori_loop(0, CHUNK, zrow, 0)
    base = s * ROWS_PT
    for i in range(5):
        pltpu.sync_copy(rows0, acc.at[pl.ds(base + i * CHUNK, CHUNK)])
    plsc.subcore_barrier()

    def gather(j, buf, sem):
        @pl.when(c == 0)
        def _():
            pltpu.async_copy(x0.at[srcv.at[j]], buf, sem)

        @pl.when(c == 1)
        def _():
            pltpu.async_copy(x1.at[srcv.at[j]], buf, sem)

    def gather_wait(buf, sem):
        pltpu.make_async_copy(x0.at[srcv.at[0]], buf, sem).wait()

    def scatter(j, buf, sem):
        pltpu.async_copy(buf, acc.at[dstv.at[j]], sem, add=True)

    def scatter_wait(buf, sem):
        pltpu.make_async_copy(buf, acc.at[dstv.at[0]], sem).wait()

    def scale(j, buf):
        def group(g, gcarry):
            wvec = wv[j, pl.ds(g * 16, 16)]
            for jj in range(16):
                e = g * 16 + jj
                w = wvec[jj]
                for k8 in range(8):
                    sl = pl.ds(k8 * 16, 16)
                    buf[e, sl] = buf[e, sl] * w
            return gcarry

        lax.fori_loop(0, CHUNK // 16, group, 0)

    def block_body(b, carry):
        pltpu.sync_copy(src_h.at[pl.ds(t0 + b * SEG, SEG)], srcv)
        pltpu.sync_copy(dst_h.at[pl.ds(t0 + b * SEG, SEG)], dstv)

        @pl.when(c == 0)
        def _():
            pltpu.sync_copy(w0_h.at[pl.ds(t0 + b * SEG, SEG)], wv)

        @pl.when(c == 1)
        def _():
            pltpu.sync_copy(w1_h.at[pl.ds(t0 + b * SEG, SEG)], wv)

        gather(0, rows0, gsem0)

        def pair_body(g, pcarry):
            j0 = 2 * g
            j1 = j0 + 1

            @pl.when(g > 0)
            def _():
                scatter_wait(rows1, ssem1)

            gather(j1, rows1, gsem1)
            gather_wait(rows0, gsem0)
            scale(j0, rows0)
            scatter(j0, rows0, ssem0)
            gather_wait(rows1, gsem1)
            scale(j1, rows1)
            scatter(j1, rows1, ssem1)
            scatter_wait(rows0, ssem0)

            @pl.when(g < SEG // 2 - 1)
            def _():
                gather(j0 + 2, rows0, gsem0)

            return pcarry

        lax.fori_loop(0, SEG // 2, pair_body, 0)
        scatter_wait(rows1, ssem1)
        return carry

    lax.fori_loop(0, NBLK, block_body, 0)
    plsc.subcore_barrier()

    @pl.when(jnp.logical_and(c == 0, s < NTILES - 1))
    def _():
        pltpu.sync_copy(acc.at[pl.ds(base, ROWS_PT)], o0.at[pl.ds(base, ROWS_PT)])

    @pl.when(jnp.logical_and(c == 1, s < NTILES - 1))
    def _():
        pltpu.sync_copy(acc.at[pl.ds(base, ROWS_PT)], o1.at[pl.ds(base, ROWS_PT)])

    @pl.when(jnp.logical_and(c == 0, s == NTILES - 1))
    def _():
        pltpu.sync_copy(acc.at[pl.ds(base, LAST_ROWS)], o0.at[pl.ds(base, LAST_ROWS)])

    @pl.when(jnp.logical_and(c == 1, s == NTILES - 1))
    def _():
        pltpu.sync_copy(acc.at[pl.ds(base, LAST_ROWS)], o1.at[pl.ds(base, LAST_ROWS)])


def _soft(x, eta):
    return jax.nn.relu(x - eta) - jax.nn.relu(-x - eta)


def kernel(F, edge_index, w0_values, w1_values, d, mask):
    dst = edge_index[0]
    src = edge_index[1]
    npad = E_PAD - E
    padidx = (jnp.arange(npad, dtype=jnp.int32) * 97) % N
    src_f = jnp.concatenate([src, padidx])
    dst_f = jnp.concatenate([dst, padidx])
    zpad = jnp.zeros((npad,), jnp.float32)
    w0_f = jnp.concatenate([w0_values, zpad])
    w1_f = jnp.concatenate([w1_values, zpad])
    order = jnp.argsort(src_f)
    src_p = src_f[order].reshape(E_PAD // CHUNK, CHUNK)
    dst_p = dst_f[order].reshape(E_PAD // CHUNK, CHUNK)
    w0_p = w0_f[order].reshape(E_PAD // CHUNK, CHUNK)
    w1_p = w1_f[order].reshape(E_PAD // CHUNK, CHUNK)

    def spmm_pair(X0, X1):
        return _spmm_pair(X0, X1, src_p, dst_p, w0_p, w1_p)

    d1 = d[:, None]
    m2 = mask * mask
    c2 = 1.0 / (d1 * m2 + GAMMA)
    c1F = d1 * m2 * F

    A0, A1 = spmm_pair(F, F)
    Y1 = jnp.zeros((N, DF), jnp.float32)
    U = F
    for k in range(1, STEPS + 1):
        v1 = Y1 - _soft(A1 + Y1, d1)
        P0, P1 = spmm_pair(A0, v1)
        U = (c1F - P1 + P0) * c2
        if k < STEPS:
            B0, B1 = spmm_pair(U, U)
            Y1 = B1 + v1
            A0, A1 = B0, B1
    return U


# speedup vs baseline: 3.0509x; 2.0374x over previous
import functools

import jax
import jax.numpy as jnp
from jax import lax
from jax.experimental import pallas as pl
from jax.experimental.pallas import tpu as pltpu
from jax.experimental.pallas import tpu_sc as plsc

N = 10000
E = 320000
DF = 128
GAMMA = 1.0
STEPS = 14

NCORES = 2
NTILES = 16
CHUNK = 64
SEG = 40
NQ = SEG // 4
NCHUNKS = 320
NBLK = NCHUNKS // SEG
EPT = NCHUNKS * CHUNK
E_PAD = EPT * NTILES
ROWS_PT = 640
ACC_N = NTILES * ROWS_PT
LAST_ROWS = N - 15 * ROWS_PT

_mesh = plsc.VectorSubcoreMesh(core_axis_name="c", subcore_axis_name="s")


@functools.partial(
    pl.kernel,
    out_type=(
        jax.ShapeDtypeStruct((N, DF), jnp.float32),
        jax.ShapeDtypeStruct((N, DF), jnp.float32),
    ),
    mesh=_mesh,
    scratch_types=[
        pltpu.VMEM_SHARED((ACC_N, DF), jnp.float32),
        pltpu.VMEM((CHUNK, DF), jnp.float32),
        pltpu.VMEM((CHUNK, DF), jnp.float32),
        pltpu.VMEM((CHUNK, DF), jnp.float32),
        pltpu.VMEM((CHUNK, DF), jnp.float32),
        pltpu.VMEM((SEG, CHUNK), jnp.int32),
        pltpu.VMEM((SEG, CHUNK), jnp.int32),
        pltpu.VMEM((SEG, CHUNK), jnp.float32),
        pltpu.SemaphoreType.DMA,
        pltpu.SemaphoreType.DMA,
        pltpu.SemaphoreType.DMA,
        pltpu.SemaphoreType.DMA,
        pltpu.SemaphoreType.DMA,
        pltpu.SemaphoreType.DMA,
        pltpu.SemaphoreType.DMA,
        pltpu.SemaphoreType.DMA,
    ],
)
def _spmm_pair(x0, x1, src_h, dst_h, w0_h, w1_h, o0, o1,
               acc, rows0, rows1, rows2, rows3, srcv, dstv, wv,
               gsem0, gsem1, gsem2, gsem3, ssem0, ssem1, ssem2, ssem3):
    c = lax.axis_index("c")
    s = lax.axis_index("s")
    t0 = s * NCHUNKS
    bufs = (rows0, rows1, rows2, rows3)
    gsems = (gsem0, gsem1, gsem2, gsem3)
    ssems = (ssem0, ssem1, ssem2, ssem3)

    zero16 = jnp.zeros((16,), jnp.float32)

    def zrow(r, carry):
        for k8 in range(8):
            rows0[r, pl.ds(k8 * 16, 16)] = zero16
        return carry

    lax.fori_loop(0, CHUNK, zrow, 0)
    base = s * ROWS_PT
    for i in range(ROWS_PT // CHUNK):
        pltpu.sync_copy(rows0, acc.at[pl.ds(base + i * CHUNK, CHUNK)])
    plsc.subcore_barrier()

    def gather(j, buf, sem):
        @pl.when(c == 0)
        def _():
            pltpu.async_copy(x0.at[srcv.at[j]], buf, sem)

        @pl.when(c == 1)
        def _():
            pltpu.async_copy(x1.at[srcv.at[j]], buf, sem)

    def gather_wait(buf, sem):
        pltpu.make_async_copy(x0.at[srcv.at[0]], buf, sem).wait()

    def scatter(j, buf, sem):
        pltpu.async_copy(buf, acc.at[dstv.at[j]], sem, add=True)

    def scatter_wait(buf, sem):
        pltpu.make_async_copy(buf, acc.at[dstv.at[0]], sem).wait()

    def scale(j, buf):
        def group(g, gcarry):
            wvec = wv[j, pl.ds(g * 16, 16)]
            for jj in range(16):
                e = g * 16 + jj
                w = wvec[jj]
                for k8 in range(8):
                    sl = pl.ds(k8 * 16, 16)
                    buf[e, sl] = buf[e, sl] * w
            return gcarry

        lax.fori_loop(0, CHUNK // 16, group, 0)

    def block_body(b, carry):
        pltpu.sync_copy(src_h.at[pl.ds(t0 + b * SEG, SEG)], srcv)
        pltpu.sync_copy(dst_h.at[pl.ds(t0 + b * SEG, SEG)], dstv)

        @pl.when(c == 0)
        def _():
            pltpu.sync_copy(w0_h.at[pl.ds(t0 + b * SEG, SEG)], wv)

        @pl.when(c == 1)
        def _():
            pltpu.sync_copy(w1_h.at[pl.ds(t0 + b * SEG, SEG)], wv)

        gather(0, bufs[0], gsems[0])
        gather(1, bufs[1], gsems[1])

        def quad_body(q, qcarry):
            for wq in range(4):
                t = 4 * q + wq
                u = wq
                u2 = (wq + 2) % 4
                gather_wait(bufs[u], gsems[u])
                scale(t, bufs[u])
                scatter(t, bufs[u], ssems[u])
                if wq < 2:
                    @pl.when(q > 0)
                    def _():
                        scatter_wait(bufs[u2], ssems[u2])

                    gather(t + 2, bufs[u2], gsems[u2])
                else:
                    @pl.when(q < NQ - 1)
                    def _():
                        scatter_wait(bufs[u2], ssems[u2])
                        gather(t + 2, bufs[u2], gsems[u2])

            return qcarry

        lax.fori_loop(0, NQ, quad_body, 0)
        for u in range(4):
            scatter_wait(bufs[u], ssems[u])
        return carry

    lax.fori_loop(0, NBLK, block_body, 0)
    plsc.subcore_barrier()

    @pl.when(jnp.logical_and(c == 0, s < NTILES - 1))
    def _():
        pltpu.sync_copy(acc.at[pl.ds(base, ROWS_PT)], o0.at[pl.ds(base, ROWS_PT)])

    @pl.when(jnp.logical_and(c == 1, s < NTILES - 1))
    def _():
        pltpu.sync_copy(acc.at[pl.ds(base, ROWS_PT)], o1.at[pl.ds(base, ROWS_PT)])

    @pl.when(jnp.logical_and(c == 0, s == NTILES - 1))
    def _():
        pltpu.sync_copy(acc.at[pl.ds(base, LAST_ROWS)], o0.at[pl.ds(base, LAST_ROWS)])

    @pl.when(jnp.logical_and(c == 1, s == NTILES - 1))
    def _():
        pltpu.sync_copy(acc.at[pl.ds(base, LAST_ROWS)], o1.at[pl.ds(base, LAST_ROWS)])


def _soft(x, eta):
    return jax.nn.relu(x - eta) - jax.nn.relu(-x - eta)


def kernel(F, edge_index, w0_values, w1_values, d, mask):
    dst = edge_index[0]
    src = edge_index[1]
    npad = E_PAD - E
    padidx = (jnp.arange(npad, dtype=jnp.int32) * 97) % N
    src_p = jnp.concatenate([src, padidx]).reshape(E_PAD // CHUNK, CHUNK)
    dst_p = jnp.concatenate([dst, padidx]).reshape(E_PAD // CHUNK, CHUNK)
    zpad = jnp.zeros((npad,), jnp.float32)
    w0_p = jnp.concatenate([w0_values, zpad]).reshape(E_PAD // CHUNK, CHUNK)
    w1_p = jnp.concatenate([w1_values, zpad]).reshape(E_PAD // CHUNK, CHUNK)

    def spmm_pair(X0, X1):
        return _spmm_pair(X0, X1, src_p, dst_p, w0_p, w1_p)

    d1 = d[:, None]
    m2 = mask * mask
    c2 = 1.0 / (d1 * m2 + GAMMA)
    c1F = d1 * m2 * F

    A0, A1 = spmm_pair(F, F)
    Y1 = jnp.zeros((N, DF), jnp.float32)
    U = F
    for k in range(1, STEPS + 1):
        v1 = Y1 - _soft(A1 + Y1, d1)
        P0, P1 = spmm_pair(A0, v1)
        U = (c1F - P1 + P0) * c2
        if k < STEPS:
            B0, B1 = spmm_pair(U, U)
            Y1 = B1 + v1
            A0, A1 = B0, B1
    return U
